# baseline (speedup 1.0000x reference)
import dataclasses
import functools

import jax
import jax.numpy as jnp
from jax import lax
from jax.experimental import pallas as pl
from jax.experimental.pallas import tpu as pltpu
from jax.experimental.pallas import tpu_sc as plsc

N = 10000
P = 10240
F = 128
BI = 256
PH = P // 2
NW = 32
RW = P // NW
GR = 8
NG = RW // GR
CW = 2048
PADC = 4096


def _sc_cparams():
    cp = pltpu.CompilerParams()
    if "needs_layout_passes" in pltpu.CompilerParams.__dataclass_fields__:
        cp = dataclasses.replace(cp, needs_layout_passes=False)
    return cp


E_EDGES = 320000
EC = E_EDGES // NW
NB = P // GR
SENT = 0x7FFFFFFF
SP = E_EDGES + 7 * NB * NW
SP_ALL = SP + 4096


def _sc_mesh():
    return plsc.VectorSubcoreMesh(core_axis_name="c", subcore_axis_name="s")


def _count_codes(codes_u):
    @functools.partial(
        pl.kernel,
        out_type=jax.ShapeDtypeStruct((NW, NB), jnp.int32),
        mesh=_sc_mesh(),
        compiler_params=_sc_cparams(),
        scratch_types=[
            pltpu.VMEM((EC,), jnp.int32),
            pltpu.VMEM((NB,), jnp.int32),
        ],
    )
    def sc_count(codes_hbm, out_hbm, codebuf, cnt):
        sid = lax.axis_index("c") * 16 + lax.axis_index("s")
        pltpu.sync_copy(codes_hbm.at[pl.ds(sid * EC, EC)], codebuf)
        z16 = jnp.zeros((16,), jnp.int32)

        @pl.loop(0, NB, step=256)
        def _z(o):
            for k in range(16):
                cnt[pl.ds(o + k * 16, 16)] = z16

        one16 = jnp.ones((16,), jnp.int32)

        @pl.loop(0, EC // 16)
        def _c(i):
            c16 = codebuf[pl.ds(pl.multiple_of(i * 16, 16), 16)]
            grp = lax.shift_right_arithmetic(c16, 17)
            plsc.addupdate_scatter(cnt, [grp], one16)

        pltpu.sync_copy(cnt, out_hbm.at[sid])

    return sc_count(codes_u)


def _place_codes(codes_u, myoff):
    @functools.partial(
        pl.kernel,
        out_type=jax.ShapeDtypeStruct((SP_ALL,), jnp.int32),
        mesh=_sc_mesh(),
        compiler_params=_sc_cparams(),
        scratch_types=[
            pltpu.VMEM((EC,), jnp.int32),
            pltpu.VMEM((NB,), jnp.int32),
            pltpu.VMEM((128,), jnp.int32),
            pltpu.VMEM((128,), jnp.int32),
            pltpu.VMEM((128,), jnp.int32),
            pltpu.VMEM((128,), jnp.int32),
            pltpu.VMEM((16,), jnp.int32),
            pltpu.VMEM((16,), jnp.int32),
            pltpu.SemaphoreType.DMA,
            pltpu.SemaphoreType.DMA,
            pltpu.SemaphoreType.DMA,
        ],
    )
    def sc_place(codes_hbm, myoff_hbm, out_hbm, codebuf, loffr, pos0, pos1,
                 val0, val1, post, valt, sem0, sem1, semt):
        sid = lax.axis_index("c") * 16 + lax.axis_index("s")
        pltpu.sync_copy(codes_hbm.at[pl.ds(sid * EC, EC)], codebuf)
        pltpu.sync_copy(myoff_hbm.at[sid], loffr)

        poss = (pos0, pos1)
        vals = (val0, val1)
        sems = (sem0, sem1)
        one16 = jnp.ones((16,), jnp.int32)

        def lanes(i, k):
            c16 = codebuf[pl.ds(pl.multiple_of(i * 16, 16), 16)]
            grp = lax.shift_right_arithmetic(c16, 17)
            cnt, _ = plsc.scan_count(grp)
            base = plsc.load_gather(loffr, [grp])
            plsc.addupdate_scatter(loffr, [grp], one16)
            return c16, base + cnt - 1

        @pl.loop(0, 78, step=2)
        def _grp(g0):
            for b in range(2):
                g = g0 + b

                @pl.when(g >= 2)
                def _wait():
                    pltpu.make_async_copy(vals[b], out_hbm.at[poss[b]],
                                          sems[b]).wait()

                for k in range(8):
                    c16, p16 = lanes(g * 8 + k, k)
                    vals[b][pl.ds(k * 16, 16)] = c16
                    poss[b][pl.ds(k * 16, 16)] = p16
                pltpu.async_copy(vals[b], out_hbm.at[poss[b]], sems[b])

        c16, p16 = lanes(624, 0)
        valt[...] = c16
        post[...] = p16
        pltpu.async_copy(valt, out_hbm.at[post], semt)
        for b in range(2):
            pltpu.make_async_copy(vals[b], out_hbm.at[poss[b]],
                                  sems[b]).wait()
        pltpu.make_async_copy(valt, out_hbm.at[post], semt).wait()

        sent16 = jnp.full((16,), jnp.int32(SENT))
        dump = jnp.int32(SP_ALL - 16) + lax.iota(jnp.int32, 16)
        for b in range(2):
            for k in range(8):
                vals[b][pl.ds(k * 16, 16)] = sent16

        @pl.loop(0, NB // 16, step=2)
        def _pad(v0):
            for b in range(2):
                v = v0 + b

                @pl.when(v >= 2)
                def _wait2():
                    pltpu.make_async_copy(vals[b], out_hbm.at[poss[b]],
                                          sems[b]).wait()

                cur = loffr[pl.ds(pl.multiple_of(v * 16, 16), 16)]
                end8 = (cur + 7) & ~7
                for r in range(7):
                    p = cur + r
                    poss[b][pl.ds(r * 16, 16)] = jnp.where(p < end8, p, dump)
                poss[b][pl.ds(112, 16)] = dump
                pltpu.async_copy(vals[b], out_hbm.at[poss[b]], sems[b])

        for b in range(2):
            pltpu.make_async_copy(vals[b], out_hbm.at[poss[b]],
                                  sems[b]).wait()

    return sc_place(codes_u, myoff)


def _build_a_packed(codes_p, gptr_p):
    mesh = plsc.VectorSubcoreMesh(core_axis_name="c", subcore_axis_name="s")

    @functools.partial(
        pl.kernel,
        out_type=jax.ShapeDtypeStruct((P, PH), jnp.int32),
        mesh=mesh,
        compiler_params=_sc_cparams(),
        scratch_types=[
            pltpu.VMEM((GR, PH), jnp.int32),
            pltpu.VMEM((GR, PH), jnp.int32),
            pltpu.VMEM((CW,), jnp.int32),
            pltpu.VMEM((48,), jnp.int32),
            pltpu.SemaphoreType.DMA,
            pltpu.SemaphoreType.DMA,
        ],
    )
    def sc_build(codes_hbm, gptr_hbm, out_hbm, acc0, acc1, codebuf, gpv,
                 sem0, sem1):
        wid = lax.axis_index("c") * 16 + lax.axis_index("s")
        wbase = wid * RW
        pltpu.sync_copy(gptr_hbm.at[pl.ds(wid * NG, 48)], gpv)

        z16 = jnp.zeros((16,), jnp.int32)
        accs = (acc0, acc1)
        sems = (sem0, sem1)

        def do_group(g, st_e, end, acc):
            @pl.loop(0, PH, step=256)
            def _z(o):
                for r in range(GR):
                    for k in range(16):
                        acc[r, pl.ds(o + k * 16, 16)] = z16

            base_row = wbase + g * GR
            st = pl.multiple_of(st_e & ~7, 8)
            nv = (end - st + 15) >> 4
            nwin = (nv + 127) >> 7

            def win_body(w2, _):
                wst = pl.multiple_of(st + w2 * CW, 8)
                pltpu.sync_copy(codes_hbm.at[pl.ds(wst, CW)], codebuf)
                mv = jnp.minimum(128, nv - w2 * 128)

                def vreg_body(i, _):
                    c16 = codebuf[pl.ds(pl.multiple_of(i * 16, 16), 16)]
                    row = lax.shift_right_arithmetic(c16, 14)
                    rl = row - base_row
                    valid = (rl >= 0) & (rl < GR)
                    col = c16 & 16383
                    wc = lax.shift_right_arithmetic(col, 1)
                    val = jnp.where((col & 1) == 1, jnp.int32(1 << 16),
                                    jnp.int32(1))
                    plsc.addupdate_scatter(acc, [rl, wc], val, mask=valid)
                    return 0

                lax.fori_loop(0, mv, vreg_body, 0)
                return 0

            lax.fori_loop(0, nwin, win_body, 0)

        @pl.loop(0, NG, step=8)
        def _chunk(g0):
            gvec = gpv[pl.ds(pl.multiple_of(g0, 8), 16)]
            for j in range(8):
                g = g0 + j
                b = j & 1

                @pl.when(g >= 2)
                def _wait():
                    pltpu.make_async_copy(
                        accs[b], out_hbm.at[pl.ds(0, GR)], sems[b]).wait()

                do_group(g, gvec[j], gvec[j + 1], accs[b])
                grow = wbase + g * GR
                pltpu.async_copy(accs[b], out_hbm.at[pl.ds(grow, GR)], sems[b])

        for b in range(2):
            pltpu.make_async_copy(accs[b], out_hbm.at[pl.ds(0, GR)],
                                  sems[b]).wait()

    return sc_build(codes_p, gptr_p)


def _split_hi_lo(h):
    hi = h.astype(jnp.bfloat16)
    lo = (h - hi.astype(jnp.float32)).astype(jnp.bfloat16)
    return hi, lo


def _layer1_body(apk_ref, hehi_ref, helo_ref, hohi_ref, holo_ref, invd_ref,
                 hself_ref, ws_ref, wn_ref, b_ref, out_ref, abf_ref):
    w = apk_ref[...]
    a_even = (w & 0xFFFF).astype(jnp.float32).astype(jnp.bfloat16)
    a_odd = lax.shift_right_arithmetic(w, 16).astype(jnp.float32).astype(
        jnp.bfloat16)
    abf_ref[:, :PH] = a_even
    abf_ref[:, PH:] = a_odd
    agg = (jnp.dot(a_even, hehi_ref[...], preferred_element_type=jnp.float32)
           + jnp.dot(a_even, helo_ref[...], preferred_element_type=jnp.float32)
           + jnp.dot(a_odd, hohi_ref[...], preferred_element_type=jnp.float32)
           + jnp.dot(a_odd, holo_ref[...], preferred_element_type=jnp.float32))
    hn = agg * invd_ref[...]
    out = (jnp.dot(hself_ref[...], ws_ref[...], preferred_element_type=jnp.float32)
           + jnp.dot(hn, wn_ref[...], preferred_element_type=jnp.float32)
           + b_ref[...])
    out_ref[...] = jnp.maximum(out, 0.0)


def _layer_body(relu, a_ref, hhi_ref, hlo_ref, invd_ref, hself_ref, ws_ref,
                wn_ref, b_ref, out_ref):
    a = a_ref[...]
    agg = (jnp.dot(a, hhi_ref[...], preferred_element_type=jnp.float32)
           + jnp.dot(a, hlo_ref[...], preferred_element_type=jnp.float32))
    hn = agg * invd_ref[...]
    out = (jnp.dot(hself_ref[...], ws_ref[...], preferred_element_type=jnp.float32)
           + jnp.dot(hn, wn_ref[...], preferred_element_type=jnp.float32)
           + b_ref[...])
    if relu:
        out = jnp.maximum(out, 0.0)
    out_ref[...] = out


_HALF = pl.BlockSpec((PH, F), lambda i: (0, 0))
_FULL = pl.BlockSpec((P, F), lambda i: (0, 0))
_ROW = pl.BlockSpec((BI, F), lambda i: (i, 0))
_W = pl.BlockSpec((F, F), lambda i: (0, 0))
_B = pl.BlockSpec((1, F), lambda i: (0, 0))
_A = pl.BlockSpec((BI, P), lambda i: (i, 0))
_APK = pl.BlockSpec((BI, PH), lambda i: (i, 0))

_CPARAMS = pltpu.CompilerParams(dimension_semantics=("parallel",))


def _layer1(Apk, xe_hi, xe_lo, xo_hi, xo_lo, invd, h, W_self, W_neigh, b):
    return pl.pallas_call(
        _layer1_body,
        grid=(P // BI,),
        in_specs=[_APK, _HALF, _HALF, _HALF, _HALF, _ROW, _ROW, _W, _W, _B],
        out_specs=[_ROW, _A],
        out_shape=[jax.ShapeDtypeStruct((P, F), jnp.float32),
                   jax.ShapeDtypeStruct((P, P), jnp.bfloat16)],
        compiler_params=_CPARAMS,
    )(Apk, xe_hi, xe_lo, xo_hi, xo_lo, invd, h, W_self, W_neigh,
      b.reshape(1, F))


def _layer(Abf, hp_hi, hp_lo, invd, h, W_self, W_neigh, b, relu):
    return pl.pallas_call(
        functools.partial(_layer_body, relu),
        grid=(P // BI,),
        in_specs=[_A, _FULL, _FULL, _ROW, _ROW, _W, _W, _B],
        out_specs=_ROW,
        out_shape=jax.ShapeDtypeStruct((P, F), jnp.float32),
        compiler_params=_CPARAMS,
    )(Abf, hp_hi, hp_lo, invd, h, W_self, W_neigh, b.reshape(1, F))


def _perm_splits(h):
    hp = jnp.concatenate([h[0::2], h[1::2]], axis=0)
    return _split_hi_lo(hp)


def kernel(x, edge_index, W_self0, W_neigh0, b0, W_self1, W_neigh1, b1,
           W_self2, W_neigh2, b2):
    src = edge_index[0].astype(jnp.int32)
    dst = edge_index[1].astype(jnp.int32)

    codes_u = (dst << 14) | src
    deg = jnp.zeros((P,), jnp.int32).at[dst].add(1)

    counts = _count_codes(codes_u)
    pc = (counts + 7) & ~7
    flat = pc.T.reshape(-1)
    cs = jnp.cumsum(flat, dtype=jnp.int32)
    starts = cs - flat
    tot = cs[-1]
    myoff = starts.reshape(NB, NW).T.copy()
    gptr = jnp.concatenate(
        [starts[::NW], jnp.full((8,), jnp.int32(0)) + tot])

    bucketed = _place_codes(codes_u, myoff)
    bucketed = jax.lax.dynamic_update_slice(
        bucketed, jnp.full((16,), jnp.int32(SENT)), (tot,))

    Apk = _build_a_packed(bucketed, gptr)

    invd = jnp.broadcast_to(
        1.0 / jnp.maximum(deg.astype(jnp.float32), 1.0)[:, None], (P, F))

    xp = jnp.pad(x, ((0, P - N), (0, 0)))
    xe_hi, xe_lo = _split_hi_lo(xp[0::2])
    xo_hi, xo_lo = _split_hi_lo(xp[1::2])

    h, Abf = _layer1(Apk, xe_hi, xe_lo, xo_hi, xo_lo, invd, xp, W_self0,
                     W_neigh0, b0)
    hp_hi, hp_lo = _perm_splits(h)
    h = _layer(Abf, hp_hi, hp_lo, invd, h, W_self1, W_neigh1, b1, relu=True)
    hp_hi, hp_lo = _perm_splits(h)
    h = _layer(Abf, hp_hi, hp_lo, invd, h, W_self2, W_neigh2, b2, relu=False)
    return h[:N]


# speedup vs baseline: 38.1864x; 38.1864x over previous
import dataclasses
import functools

import jax
import jax.numpy as jnp
from jax import lax
from jax.experimental import pallas as pl
from jax.experimental.pallas import tpu as pltpu
from jax.experimental.pallas import tpu_sc as plsc

N = 10000
P = 10240
F = 128
BI = 256
PH = P // 2
NW = 32
RW = P // NW
GR = 8
NG = RW // GR
CW = 2048
PADC = 4096


def _sc_cparams():
    cp = pltpu.CompilerParams()
    if "needs_layout_passes" in pltpu.CompilerParams.__dataclass_fields__:
        cp = dataclasses.replace(cp, needs_layout_passes=False)
    return cp


def _build_a_packed(codes_p, gptr_p):
    mesh = plsc.VectorSubcoreMesh(core_axis_name="c", subcore_axis_name="s")

    @functools.partial(
        pl.kernel,
        out_type=jax.ShapeDtypeStruct((P, PH), jnp.int32),
        mesh=mesh,
        compiler_params=_sc_cparams(),
        scratch_types=[
            pltpu.VMEM((GR, PH), jnp.int32),
            pltpu.VMEM((GR, PH), jnp.int32),
            pltpu.VMEM((CW,), jnp.int32),
            pltpu.VMEM((48,), jnp.int32),
            pltpu.SemaphoreType.DMA,
            pltpu.SemaphoreType.DMA,
        ],
    )
    def sc_build(codes_hbm, gptr_hbm, out_hbm, acc0, acc1, codebuf, gpv,
                 sem0, sem1):
        wid = lax.axis_index("c") * 16 + lax.axis_index("s")
        wbase = wid * RW
        pltpu.sync_copy(gptr_hbm.at[pl.ds(wid * NG, 48)], gpv)

        z16 = jnp.zeros((16,), jnp.int32)
        accs = (acc0, acc1)
        sems = (sem0, sem1)

        def do_group(g, st_e, end, acc):
            @pl.loop(0, PH, step=256)
            def _z(o):
                for r in range(GR):
                    for k in range(16):
                        acc[r, pl.ds(o + k * 16, 16)] = z16

            base_row = wbase + g * GR
            st = pl.multiple_of(st_e & ~7, 8)
            nv = (end - st + 15) >> 4
            nwin = (nv + 127) >> 7

            def win_body(w2, _):
                wst = pl.multiple_of(st + w2 * CW, 8)
                pltpu.sync_copy(codes_hbm.at[pl.ds(wst, CW)], codebuf)
                mv = jnp.minimum(128, nv - w2 * 128)

                def vreg_body(i, _):
                    c16 = codebuf[pl.ds(pl.multiple_of(i * 16, 16), 16)]
                    row = lax.shift_right_arithmetic(c16, 14)
                    rl = row - base_row
                    valid = (rl >= 0) & (rl < GR)
                    col = c16 & 16383
                    wc = lax.shift_right_arithmetic(col, 1)
                    val = jnp.where((col & 1) == 1, jnp.int32(1 << 16),
                                    jnp.int32(1))
                    plsc.addupdate_scatter(acc, [rl, wc], val, mask=valid)
                    return 0

                lax.fori_loop(0, mv, vreg_body, 0)
                return 0

            lax.fori_loop(0, nwin, win_body, 0)

        @pl.loop(0, NG, step=8)
        def _chunk(g0):
            gvec = gpv[pl.ds(pl.multiple_of(g0, 8), 16)]
            for j in range(8):
                g = g0 + j
                b = j & 1

                @pl.when(g >= 2)
                def _wait():
                    pltpu.make_async_copy(
                        accs[b], out_hbm.at[pl.ds(0, GR)], sems[b]).wait()

                do_group(g, gvec[j], gvec[j + 1], accs[b])
                grow = wbase + g * GR
                pltpu.async_copy(accs[b], out_hbm.at[pl.ds(grow, GR)], sems[b])

        for b in range(2):
            pltpu.make_async_copy(accs[b], out_hbm.at[pl.ds(0, GR)],
                                  sems[b]).wait()

    return sc_build(codes_p, gptr_p)


def _split_hi_lo(h):
    hi = h.astype(jnp.bfloat16)
    lo = (h - hi.astype(jnp.float32)).astype(jnp.bfloat16)
    return hi, lo


def _layer1_body(apk_ref, hehi_ref, helo_ref, hohi_ref, holo_ref, invd_ref,
                 hself_ref, ws_ref, wn_ref, b_ref, out_ref, abf_ref):
    w = apk_ref[...]
    a_even = (w & 0xFFFF).astype(jnp.float32).astype(jnp.bfloat16)
    a_odd = lax.shift_right_arithmetic(w, 16).astype(jnp.float32).astype(
        jnp.bfloat16)
    abf_ref[:, :PH] = a_even
    abf_ref[:, PH:] = a_odd
    agg = (jnp.dot(a_even, hehi_ref[...], preferred_element_type=jnp.float32)
           + jnp.dot(a_even, helo_ref[...], preferred_element_type=jnp.float32)
           + jnp.dot(a_odd, hohi_ref[...], preferred_element_type=jnp.float32)
           + jnp.dot(a_odd, holo_ref[...], preferred_element_type=jnp.float32))
    hn = agg * invd_ref[...]
    out = (jnp.dot(hself_ref[...], ws_ref[...], preferred_element_type=jnp.float32)
           + jnp.dot(hn, wn_ref[...], preferred_element_type=jnp.float32)
           + b_ref[...])
    out_ref[...] = jnp.maximum(out, 0.0)


def _layer_body(relu, a_ref, hhi_ref, hlo_ref, invd_ref, hself_ref, ws_ref,
                wn_ref, b_ref, out_ref):
    a = a_ref[...]
    agg = (jnp.dot(a, hhi_ref[...], preferred_element_type=jnp.float32)
           + jnp.dot(a, hlo_ref[...], preferred_element_type=jnp.float32))
    hn = agg * invd_ref[...]
    out = (jnp.dot(hself_ref[...], ws_ref[...], preferred_element_type=jnp.float32)
           + jnp.dot(hn, wn_ref[...], preferred_element_type=jnp.float32)
           + b_ref[...])
    if relu:
        out = jnp.maximum(out, 0.0)
    out_ref[...] = out


_HALF = pl.BlockSpec((PH, F), lambda i: (0, 0))
_FULL = pl.BlockSpec((P, F), lambda i: (0, 0))
_ROW = pl.BlockSpec((BI, F), lambda i: (i, 0))
_W = pl.BlockSpec((F, F), lambda i: (0, 0))
_B = pl.BlockSpec((1, F), lambda i: (0, 0))
_A = pl.BlockSpec((BI, P), lambda i: (i, 0))
_APK = pl.BlockSpec((BI, PH), lambda i: (i, 0))

_CPARAMS = pltpu.CompilerParams(dimension_semantics=("parallel",))


def _layer1(Apk, xe_hi, xe_lo, xo_hi, xo_lo, invd, h, W_self, W_neigh, b):
    return pl.pallas_call(
        _layer1_body,
        grid=(P // BI,),
        in_specs=[_APK, _HALF, _HALF, _HALF, _HALF, _ROW, _ROW, _W, _W, _B],
        out_specs=[_ROW, _A],
        out_shape=[jax.ShapeDtypeStruct((P, F), jnp.float32),
                   jax.ShapeDtypeStruct((P, P), jnp.bfloat16)],
        compiler_params=_CPARAMS,
    )(Apk, xe_hi, xe_lo, xo_hi, xo_lo, invd, h, W_self, W_neigh,
      b.reshape(1, F))


def _layer(Abf, hp_hi, hp_lo, invd, h, W_self, W_neigh, b, relu):
    return pl.pallas_call(
        functools.partial(_layer_body, relu),
        grid=(P // BI,),
        in_specs=[_A, _FULL, _FULL, _ROW, _ROW, _W, _W, _B],
        out_specs=_ROW,
        out_shape=jax.ShapeDtypeStruct((P, F), jnp.float32),
        compiler_params=_CPARAMS,
    )(Abf, hp_hi, hp_lo, invd, h, W_self, W_neigh, b.reshape(1, F))


def _perm_splits(h):
    hp = jnp.concatenate([h[0::2], h[1::2]], axis=0)
    return _split_hi_lo(hp)


def kernel(x, edge_index, W_self0, W_neigh0, b0, W_self1, W_neigh1, b1,
           W_self2, W_neigh2, b2):
    src = edge_index[0].astype(jnp.int32)
    dst = edge_index[1].astype(jnp.int32)
    E = src.shape[0]

    codes = lax.sort((dst << 14) | src, is_stable=False)
    codes_p = jnp.concatenate(
        [codes, jnp.full((PADC,), jnp.int32(0x7FFFFFFF))])
    deg = jnp.zeros((P,), jnp.int32).at[dst].add(1)
    rowptr = jnp.concatenate(
        [jnp.zeros((1,), jnp.int32), jnp.cumsum(deg, dtype=jnp.int32)])
    gptr = rowptr[::GR]
    gptr_p = jnp.concatenate([gptr, jnp.full((7,), jnp.int32(E))])

    Apk = _build_a_packed(codes_p, gptr_p)

    invd = jnp.broadcast_to(
        1.0 / jnp.maximum(deg.astype(jnp.float32), 1.0)[:, None], (P, F))

    xp = jnp.pad(x, ((0, P - N), (0, 0)))
    xe_hi, xe_lo = _split_hi_lo(xp[0::2])
    xo_hi, xo_lo = _split_hi_lo(xp[1::2])

    h, Abf = _layer1(Apk, xe_hi, xe_lo, xo_hi, xo_lo, invd, xp, W_self0,
                     W_neigh0, b0)
    hp_hi, hp_lo = _perm_splits(h)
    h = _layer(Abf, hp_hi, hp_lo, invd, h, W_self1, W_neigh1, b1, relu=True)
    hp_hi, hp_lo = _perm_splits(h)
    h = _layer(Abf, hp_hi, hp_lo, invd, h, W_self2, W_neigh2, b2, relu=False)
    return h[:N]


# speedup vs baseline: 39.7365x; 1.0406x over previous
import dataclasses
import functools

import jax
import jax.numpy as jnp
from jax import lax
from jax.experimental import pallas as pl
from jax.experimental.pallas import tpu as pltpu
from jax.experimental.pallas import tpu_sc as plsc

N = 10000
P = 10240
F = 128
BI = 512
PH = P // 2
NW = 32
RW = P // NW
GR = 8
NG = RW // GR
CW = 2048
PADC = 4096


def _sc_cparams():
    cp = pltpu.CompilerParams()
    if "needs_layout_passes" in pltpu.CompilerParams.__dataclass_fields__:
        cp = dataclasses.replace(cp, needs_layout_passes=False)
    return cp


def _build_a_packed(codes_p, gptr_p):
    mesh = plsc.VectorSubcoreMesh(core_axis_name="c", subcore_axis_name="s")

    @functools.partial(
        pl.kernel,
        out_type=jax.ShapeDtypeStruct((P, PH), jnp.int32),
        mesh=mesh,
        compiler_params=_sc_cparams(),
        scratch_types=[
            pltpu.VMEM((GR, PH), jnp.int32),
            pltpu.VMEM((GR, PH), jnp.int32),
            pltpu.VMEM((CW,), jnp.int32),
            pltpu.VMEM((48,), jnp.int32),
            pltpu.SemaphoreType.DMA,
            pltpu.SemaphoreType.DMA,
        ],
    )
    def sc_build(codes_hbm, gptr_hbm, out_hbm, acc0, acc1, codebuf, gpv,
                 sem0, sem1):
        wid = lax.axis_index("c") * 16 + lax.axis_index("s")
        wbase = wid * RW
        pltpu.sync_copy(gptr_hbm.at[pl.ds(wid * NG, 48)], gpv)

        z16 = jnp.zeros((16,), jnp.int32)
        accs = (acc0, acc1)
        sems = (sem0, sem1)

        def do_group(g, st_e, end, acc):
            @pl.loop(0, PH, step=256)
            def _z(o):
                for r in range(GR):
                    for k in range(16):
                        acc[r, pl.ds(o + k * 16, 16)] = z16

            base_row = wbase + g * GR
            st = pl.multiple_of(st_e & ~7, 8)
            nv = (end - st + 15) >> 4
            nwin = (nv + 127) >> 7

            def win_body(w2, _):
                wst = pl.multiple_of(st + w2 * CW, 8)
                pltpu.sync_copy(codes_hbm.at[pl.ds(wst, CW)], codebuf)
                mv = jnp.minimum(128, nv - w2 * 128)

                def vreg_body(i, _):
                    c16 = codebuf[pl.ds(pl.multiple_of(i * 16, 16), 16)]
                    row = lax.shift_right_arithmetic(c16, 14)
                    rl = row - base_row
                    valid = (rl >= 0) & (rl < GR)
                    col = c16 & 16383
                    wc = lax.shift_right_arithmetic(col, 1)
                    val = jnp.where((col & 1) == 1, jnp.int32(1 << 16),
                                    jnp.int32(1))
                    plsc.addupdate_scatter(acc, [rl, wc], val, mask=valid)
                    return 0

                lax.fori_loop(0, mv, vreg_body, 0)
                return 0

            lax.fori_loop(0, nwin, win_body, 0)

        @pl.loop(0, NG, step=8)
        def _chunk(g0):
            gvec = gpv[pl.ds(pl.multiple_of(g0, 8), 16)]
            for j in range(8):
                g = g0 + j
                b = j & 1

                @pl.when(g >= 2)
                def _wait():
                    pltpu.make_async_copy(
                        accs[b], out_hbm.at[pl.ds(0, GR)], sems[b]).wait()

                do_group(g, gvec[j], gvec[j + 1], accs[b])
                grow = wbase + g * GR
                pltpu.async_copy(accs[b], out_hbm.at[pl.ds(grow, GR)], sems[b])

        for b in range(2):
            pltpu.make_async_copy(accs[b], out_hbm.at[pl.ds(0, GR)],
                                  sems[b]).wait()

    return sc_build(codes_p, gptr_p)


def _split_hi_lo(h):
    hi = h.astype(jnp.bfloat16)
    lo = (h - hi.astype(jnp.float32)).astype(jnp.bfloat16)
    return hi, lo


def _layer1_body(apk_ref, hehi_ref, helo_ref, hohi_ref, holo_ref, invd_ref,
                 hself_ref, ws_ref, wn_ref, b_ref, out_ref, abf_ref):
    w = apk_ref[...]
    a_even = (w & 0xFFFF).astype(jnp.float32).astype(jnp.bfloat16)
    a_odd = lax.shift_right_arithmetic(w, 16).astype(jnp.float32).astype(
        jnp.bfloat16)
    abf_ref[:, :PH] = a_even
    abf_ref[:, PH:] = a_odd
    agg = (jnp.dot(a_even, hehi_ref[...], preferred_element_type=jnp.float32)
           + jnp.dot(a_even, helo_ref[...], preferred_element_type=jnp.float32)
           + jnp.dot(a_odd, hohi_ref[...], preferred_element_type=jnp.float32)
           + jnp.dot(a_odd, holo_ref[...], preferred_element_type=jnp.float32))
    hn = agg * invd_ref[...]
    out = (jnp.dot(hself_ref[...], ws_ref[...], preferred_element_type=jnp.float32)
           + jnp.dot(hn, wn_ref[...], preferred_element_type=jnp.float32)
           + b_ref[...])
    out_ref[...] = jnp.maximum(out, 0.0)


def _layer_body(relu, a_ref, hhi_ref, hlo_ref, invd_ref, hself_ref, ws_ref,
                wn_ref, b_ref, out_ref):
    a = a_ref[...]
    agg = (jnp.dot(a, hhi_ref[...], preferred_element_type=jnp.float32)
           + jnp.dot(a, hlo_ref[...], preferred_element_type=jnp.float32))
    hn = agg * invd_ref[...]
    out = (jnp.dot(hself_ref[...], ws_ref[...], preferred_element_type=jnp.float32)
           + jnp.dot(hn, wn_ref[...], preferred_element_type=jnp.float32)
           + b_ref[...])
    if relu:
        out = jnp.maximum(out, 0.0)
    out_ref[...] = out


_HALF = pl.BlockSpec((PH, F), lambda i: (0, 0))
_FULL = pl.BlockSpec((P, F), lambda i: (0, 0))
_ROW = pl.BlockSpec((BI, F), lambda i: (i, 0))
_W = pl.BlockSpec((F, F), lambda i: (0, 0))
_B = pl.BlockSpec((1, F), lambda i: (0, 0))
_A = pl.BlockSpec((BI, P), lambda i: (i, 0))
_APK = pl.BlockSpec((BI, PH), lambda i: (i, 0))

_CPARAMS = pltpu.CompilerParams(dimension_semantics=("parallel",))


def _layer1(Apk, xe_hi, xe_lo, xo_hi, xo_lo, invd, h, W_self, W_neigh, b):
    return pl.pallas_call(
        _layer1_body,
        grid=(P // BI,),
        in_specs=[_APK, _HALF, _HALF, _HALF, _HALF, _ROW, _ROW, _W, _W, _B],
        out_specs=[_ROW, _A],
        out_shape=[jax.ShapeDtypeStruct((P, F), jnp.float32),
                   jax.ShapeDtypeStruct((P, P), jnp.bfloat16)],
        compiler_params=_CPARAMS,
    )(Apk, xe_hi, xe_lo, xo_hi, xo_lo, invd, h, W_self, W_neigh,
      b.reshape(1, F))


def _layer(Abf, hp_hi, hp_lo, invd, h, W_self, W_neigh, b, relu):
    return pl.pallas_call(
        functools.partial(_layer_body, relu),
        grid=(P // BI,),
        in_specs=[_A, _FULL, _FULL, _ROW, _ROW, _W, _W, _B],
        out_specs=_ROW,
        out_shape=jax.ShapeDtypeStruct((P, F), jnp.float32),
        compiler_params=_CPARAMS,
    )(Abf, hp_hi, hp_lo, invd, h, W_self, W_neigh, b.reshape(1, F))


def _perm_splits(h):
    hp = jnp.concatenate([h[0::2], h[1::2]], axis=0)
    return _split_hi_lo(hp)


def kernel(x, edge_index, W_self0, W_neigh0, b0, W_self1, W_neigh1, b1,
           W_self2, W_neigh2, b2):
    src = edge_index[0].astype(jnp.int32)
    dst = edge_index[1].astype(jnp.int32)
    E = src.shape[0]

    codes = lax.sort((dst << 14) | src, is_stable=False)
    codes_p = jnp.concatenate(
        [codes, jnp.full((PADC,), jnp.int32(0x7FFFFFFF))])
    deg = jnp.zeros((P,), jnp.int32).at[dst].add(1)
    rowptr = jnp.concatenate(
        [jnp.zeros((1,), jnp.int32), jnp.cumsum(deg, dtype=jnp.int32)])
    gptr = rowptr[::GR]
    gptr_p = jnp.concatenate([gptr, jnp.full((7,), jnp.int32(E))])

    Apk = _build_a_packed(codes_p, gptr_p)

    invd = jnp.broadcast_to(
        1.0 / jnp.maximum(deg.astype(jnp.float32), 1.0)[:, None], (P, F))

    xp = jnp.pad(x, ((0, P - N), (0, 0)))
    xe_hi, xe_lo = _split_hi_lo(xp[0::2])
    xo_hi, xo_lo = _split_hi_lo(xp[1::2])

    h, Abf = _layer1(Apk, xe_hi, xe_lo, xo_hi, xo_lo, invd, xp, W_self0,
                     W_neigh0, b0)
    hp_hi, hp_lo = _perm_splits(h)
    h = _layer(Abf, hp_hi, hp_lo, invd, h, W_self1, W_neigh1, b1, relu=True)
    hp_hi, hp_lo = _perm_splits(h)
    h = _layer(Abf, hp_hi, hp_lo, invd, h, W_self2, W_neigh2, b2, relu=False)
    return h[:N]
